# Initial kernel scaffold; baseline (speedup 1.0000x reference)
import jax
import jax.numpy as jnp
from jax.experimental import pallas as pl


def kernel(h, e, edge_index, WQ, WK, WV, We, WOh, bOh, WOe, bOe, g1h, b1h, g1e, b1e, Wf0, bf0, Wf1, bf1):
    raise NotImplementedError("write your pallas kernel here")


# speedup vs baseline: 5.6402x; 5.6402x over previous
import functools

import jax
import jax.numpy as jnp
import numpy as np
from jax.experimental import pallas as pl
from jax.experimental.pallas import tpu as pltpu

N, E, D, H, DH = 10000, 320000, 128, 8, 16
NB = 1000
EB = 512
SCALE = 1.0 / np.sqrt(DH).astype(np.float32)

_S16 = np.zeros((D, 16), np.float32)
for _h in range(H):
    _S16[_h * DH:(_h + 1) * DH, _h] = 1.0
_T16 = _S16.T.copy()


def _qkv_body(h_ref, wq_ref, wk_ref, wv_ref, q_ref, k_ref, v_ref):
    hb = h_ref[...]
    q_ref[...] = jnp.dot(hb, wq_ref[...], preferred_element_type=jnp.float32)
    k_ref[...] = jnp.dot(hb, wk_ref[...], preferred_element_type=jnp.float32)
    v_ref[...] = jnp.dot(hb, wv_ref[...], preferred_element_type=jnp.float32)


def _pass1_body(e_ref, kg_ref, qg_ref, vg_ref, we_ref, woe_ref, boe_ref,
                g1e_ref, b1e_ref, s16_ref, t16_ref,
                e2_ref, vsc_ref, scz_ref):
    pe = jnp.dot(e_ref[...], we_ref[...], preferred_element_type=jnp.float32)
    score = kg_ref[...] * qg_ref[...] * pe * SCALE
    sc16 = jnp.exp(jnp.clip(
        jnp.dot(score, s16_ref[...], preferred_element_type=jnp.float32),
        -5.0, 5.0))
    vsc_ref[...] = vg_ref[...] * jnp.dot(
        sc16, t16_ref[...], preferred_element_type=jnp.float32)
    scz_ref[...] = sc16
    t = jnp.dot(score, woe_ref[...], preferred_element_type=jnp.float32) \
        + boe_ref[...]
    m = jnp.mean(t, axis=1, keepdims=True)
    v = jnp.mean((t - m) * (t - m), axis=1, keepdims=True)
    e2_ref[...] = (t - m) * jax.lax.rsqrt(v + 1e-5) * g1e_ref[...] \
        + b1e_ref[...]


def _h2_body(wv_ref, z_ref, t16_ref, woh_ref, boh_ref, g1h_ref, b1h_ref,
             wtop_ref, wbot_ref, a_ref, b_ref):
    zb = jnp.dot(z_ref[...], t16_ref[...],
                 preferred_element_type=jnp.float32) + 1e-6
    h_out = wv_ref[...] / zb
    t = jnp.dot(h_out, woh_ref[...], preferred_element_type=jnp.float32) \
        + boh_ref[...]
    m = jnp.mean(t, axis=1, keepdims=True)
    v = jnp.mean((t - m) * (t - m), axis=1, keepdims=True)
    h2 = (t - m) * jax.lax.rsqrt(v + 1e-5) * g1h_ref[...] + b1h_ref[...]
    a_ref[...] = jnp.dot(h2, wtop_ref[...], preferred_element_type=jnp.float32)
    b_ref[...] = jnp.dot(h2, wbot_ref[...], preferred_element_type=jnp.float32)


def _final_body(ag_ref, bg_ref, e2_ref, wmid_ref, bf0_ref, wf1_ref, bf1_ref,
                out_ref):
    y = ag_ref[...] + bg_ref[...] + jnp.dot(
        e2_ref[...], wmid_ref[...], preferred_element_type=jnp.float32) \
        + bf0_ref[...]
    y = jnp.maximum(y, 0.0)
    out_ref[...] = jnp.dot(y, wf1_ref[...],
                           preferred_element_type=jnp.float32) + bf1_ref[...]


def _row(x):
    return x.reshape(1, -1)


def kernel(h, e, edge_index, WQ, WK, WV, We, WOh, bOh, WOe, bOe,
           g1h, b1h, g1e, b1e, Wf0, bf0, Wf1, bf1):
    src = edge_index[0]
    dst = edge_index[1]
    s16 = jnp.asarray(_S16)
    t16 = jnp.asarray(_T16)

    wspec = pl.BlockSpec((D, D), lambda i: (0, 0))
    q_t, k_t, v_t = pl.pallas_call(
        _qkv_body,
        grid=(N // NB,),
        in_specs=[pl.BlockSpec((NB, D), lambda i: (i, 0)), wspec, wspec,
                  wspec],
        out_specs=[pl.BlockSpec((NB, D), lambda i: (i, 0))] * 3,
        out_shape=[jax.ShapeDtypeStruct((N, D), jnp.float32)] * 3,
    )(h, WQ, WK, WV)

    kg = jnp.take(k_t, src, axis=0)
    qg = jnp.take(q_t, dst, axis=0)
    vg = jnp.take(v_t, src, axis=0)

    nb_e = E // EB
    ebs = pl.BlockSpec((EB, D), lambda i: (i, 0))
    e2, vsc, scz = pl.pallas_call(
        _pass1_body,
        grid=(nb_e,),
        in_specs=[
            pl.BlockSpec((EB, D + 2), lambda i: (i, 0)),
            ebs, ebs, ebs,
            pl.BlockSpec((D + 2, D), lambda i: (0, 0)),
            pl.BlockSpec((D, D), lambda i: (0, 0)),
            pl.BlockSpec((1, D), lambda i: (0, 0)),
            pl.BlockSpec((1, D), lambda i: (0, 0)),
            pl.BlockSpec((1, D), lambda i: (0, 0)),
            pl.BlockSpec((D, 16), lambda i: (0, 0)),
            pl.BlockSpec((16, D), lambda i: (0, 0)),
        ],
        out_specs=[ebs, ebs, pl.BlockSpec((EB, 16), lambda i: (i, 0))],
        out_shape=[
            jax.ShapeDtypeStruct((E, D), jnp.float32),
            jax.ShapeDtypeStruct((E, D), jnp.float32),
            jax.ShapeDtypeStruct((E, 16), jnp.float32),
        ],
    )(e, kg, qg, vg, We, WOe, _row(bOe), _row(g1e), _row(b1e), s16, t16)

    wv = jax.ops.segment_sum(vsc, dst, num_segments=N)
    z = jax.ops.segment_sum(scz, dst, num_segments=N)

    a_t, b_t = pl.pallas_call(
        _h2_body,
        grid=(N // NB,),
        in_specs=[
            pl.BlockSpec((NB, D), lambda i: (i, 0)),
            pl.BlockSpec((NB, 16), lambda i: (i, 0)),
            pl.BlockSpec((16, D), lambda i: (0, 0)),
            pl.BlockSpec((D, D), lambda i: (0, 0)),
            pl.BlockSpec((1, D), lambda i: (0, 0)),
            pl.BlockSpec((1, D), lambda i: (0, 0)),
            pl.BlockSpec((1, D), lambda i: (0, 0)),
            pl.BlockSpec((D, 192), lambda i: (0, 0)),
            pl.BlockSpec((D, 192), lambda i: (0, 0)),
        ],
        out_specs=[pl.BlockSpec((NB, 192), lambda i: (i, 0))] * 2,
        out_shape=[jax.ShapeDtypeStruct((N, 192), jnp.float32)] * 2,
    )(wv, z, t16, WOh, _row(bOh), _row(g1h), _row(b1h),
      Wf0[0:D], Wf0[2 * D:3 * D])

    ag = jnp.take(a_t, src, axis=0)
    bg = jnp.take(b_t, dst, axis=0)

    out = pl.pallas_call(
        _final_body,
        grid=(nb_e,),
        in_specs=[
            pl.BlockSpec((EB, 192), lambda i: (i, 0)),
            pl.BlockSpec((EB, 192), lambda i: (i, 0)),
            ebs,
            pl.BlockSpec((D, 192), lambda i: (0, 0)),
            pl.BlockSpec((1, 192), lambda i: (0, 0)),
            pl.BlockSpec((192, 4), lambda i: (0, 0)),
            pl.BlockSpec((1, 4), lambda i: (0, 0)),
        ],
        out_specs=pl.BlockSpec((EB, 4), lambda i: (i, 0)),
        out_shape=jax.ShapeDtypeStruct((E, 4), jnp.float32),
    )(ag, bg, e2, Wf0[D:2 * D], _row(bf0), Wf1, _row(bf1))
    return out


# speedup vs baseline: 10.8275x; 1.9197x over previous
import functools

import jax
import jax.numpy as jnp
import numpy as np
from jax.experimental import pallas as pl
from jax.experimental.pallas import tpu as pltpu
from jax.experimental.pallas import tpu_sc as plsc

N, E, D, H, DH = 10000, 320000, 128, 8, 16
NB = 1000
EB = 512
SCALE = 1.0 / np.sqrt(DH).astype(np.float32)

_S16 = np.zeros((D, 16), np.float32)
for _h in range(H):
    _S16[_h * DH:(_h + 1) * DH, _h] = 1.0
_T16 = _S16.T.copy()


def _qkv_body(h_ref, wq_ref, wk_ref, wv_ref, q_ref, k_ref, v_ref):
    hb = h_ref[...]
    q_ref[...] = jnp.dot(hb, wq_ref[...], preferred_element_type=jnp.float32)
    k_ref[...] = jnp.dot(hb, wk_ref[...], preferred_element_type=jnp.float32)
    v_ref[...] = jnp.dot(hb, wv_ref[...], preferred_element_type=jnp.float32)


def _pass1_body(e_ref, kg_ref, qg_ref, vg_ref, we_ref, woe_ref, boe_ref,
                g1e_ref, b1e_ref, s16_ref, t16_ref,
                e2_ref, vsc_ref, scz_ref):
    pe = jnp.dot(e_ref[...], we_ref[...], preferred_element_type=jnp.float32)
    score = kg_ref[...] * qg_ref[...] * pe * SCALE
    sc16 = jnp.exp(jnp.clip(
        jnp.dot(score, s16_ref[...], preferred_element_type=jnp.float32),
        -5.0, 5.0))
    vsc_ref[...] = vg_ref[...] * jnp.dot(
        sc16, t16_ref[...], preferred_element_type=jnp.float32)
    scz_ref[...] = sc16
    t = jnp.dot(score, woe_ref[...], preferred_element_type=jnp.float32) \
        + boe_ref[...]
    m = jnp.mean(t, axis=1, keepdims=True)
    v = jnp.mean((t - m) * (t - m), axis=1, keepdims=True)
    e2_ref[...] = (t - m) * jax.lax.rsqrt(v + 1e-5) * g1e_ref[...] \
        + b1e_ref[...]


def _h2_body(wv_ref, z_ref, t16_ref, woh_ref, boh_ref, g1h_ref, b1h_ref,
             h2_ref):
    zb = jnp.dot(z_ref[...], t16_ref[...],
                 preferred_element_type=jnp.float32) + 1e-6
    h_out = wv_ref[...] / zb
    t = jnp.dot(h_out, woh_ref[...], preferred_element_type=jnp.float32) \
        + boh_ref[...]
    m = jnp.mean(t, axis=1, keepdims=True)
    v = jnp.mean((t - m) * (t - m), axis=1, keepdims=True)
    h2_ref[...] = (t - m) * jax.lax.rsqrt(v + 1e-5) * g1h_ref[...] \
        + b1h_ref[...]


def _final_body(hs_ref, hd_ref, e2_ref, wtop_ref, wmid_ref, wbot_ref,
                bf0_ref, wf1_ref, bf1_ref, out_ref):
    y = (jnp.dot(hs_ref[...], wtop_ref[...],
                 preferred_element_type=jnp.float32)
         + jnp.dot(e2_ref[...], wmid_ref[...],
                   preferred_element_type=jnp.float32)
         + jnp.dot(hd_ref[...], wbot_ref[...],
                   preferred_element_type=jnp.float32)
         + bf0_ref[...])
    y = jnp.maximum(y, 0.0)
    out_ref[...] = jnp.dot(y, wf1_ref[...],
                           preferred_element_type=jnp.float32) + bf1_ref[...]


def _row(x):
    return x.reshape(1, -1)


_NC, _NS = 2, 16
_NW = _NC * _NS
_RPW = E // _NW
_CH = 80
_NCH = _RPW // _CH


def _sc_mesh():
    return plsc.VectorSubcoreMesh(core_axis_name="c", subcore_axis_name="s",
                                  num_cores=_NC, num_subcores=_NS)


def _make_gather_body(sels):
    n_t = len(sels)

    def body(*refs):
        tables = refs[:n_t]
        src2, dst2 = refs[n_t], refs[n_t + 1]
        outs = refs[n_t + 2:2 * n_t + 2]
        idx_s, idx_d = refs[2 * n_t + 2], refs[2 * n_t + 3]
        bufs = refs[2 * n_t + 4:3 * n_t + 4]
        sems = refs[3 * n_t + 4:]
        w = jax.lax.axis_index("s") * _NC + jax.lax.axis_index("c")
        base = w * _NCH
        pltpu.sync_copy(src2.at[w], idx_s)
        pltpu.sync_copy(dst2.at[w], idx_d)
        idxs = (idx_s, idx_d)

        def step(j, carry):
            cps = [pltpu.async_copy(tables[t].at[idxs[sels[t]].at[j]],
                                    bufs[t], sems[t]) for t in range(n_t)]
            for cp in cps:
                cp.wait()
            row0 = (base + j) * _CH
            for t in range(n_t):
                pltpu.sync_copy(bufs[t], outs[t].at[pl.ds(row0, _CH)])
            return carry

        jax.lax.fori_loop(0, _NCH, step, 0)

    return body


def _sc_gather(tables, sels, src2, dst2):
    n_t = len(tables)
    d_cols = [int(t.shape[1]) for t in tables]
    scratch = (
        [pltpu.VMEM((_NCH, _CH), jnp.int32)] * 2
        + [pltpu.VMEM((_CH, d_cols[t]), jnp.float32) for t in range(n_t)]
        + [pltpu.SemaphoreType.DMA] * n_t)
    k = pl.kernel(
        _make_gather_body(sels),
        out_type=[jax.ShapeDtypeStruct((E, d_cols[t]), jnp.float32)
                  for t in range(n_t)],
        mesh=_sc_mesh(),
        scratch_types=scratch,
    )
    return k(*tables, src2, dst2)


def kernel(h, e, edge_index, WQ, WK, WV, We, WOh, bOh, WOe, bOe,
           g1h, b1h, g1e, b1e, Wf0, bf0, Wf1, bf1):
    src = edge_index[0]
    dst = edge_index[1]
    src2 = src.reshape(_NW, _NCH, _CH)
    dst2 = dst.reshape(_NW, _NCH, _CH)
    s16 = jnp.asarray(_S16)
    t16 = jnp.asarray(_T16)

    wspec = pl.BlockSpec((D, D), lambda i: (0, 0))
    q_t, k_t, v_t = pl.pallas_call(
        _qkv_body,
        grid=(N // NB,),
        in_specs=[pl.BlockSpec((NB, D), lambda i: (i, 0)), wspec, wspec,
                  wspec],
        out_specs=[pl.BlockSpec((NB, D), lambda i: (i, 0))] * 3,
        out_shape=[jax.ShapeDtypeStruct((N, D), jnp.float32)] * 3,
    )(h, WQ, WK, WV)

    kg, qg, vg = _sc_gather((k_t, q_t, v_t), (0, 1, 0), src2, dst2)

    nb_e = E // EB
    ebs = pl.BlockSpec((EB, D), lambda i: (i, 0))
    e2, vsc, scz = pl.pallas_call(
        _pass1_body,
        grid=(nb_e,),
        in_specs=[
            pl.BlockSpec((EB, D + 2), lambda i: (i, 0)),
            ebs, ebs, ebs,
            pl.BlockSpec((D + 2, D), lambda i: (0, 0)),
            pl.BlockSpec((D, D), lambda i: (0, 0)),
            pl.BlockSpec((1, D), lambda i: (0, 0)),
            pl.BlockSpec((1, D), lambda i: (0, 0)),
            pl.BlockSpec((1, D), lambda i: (0, 0)),
            pl.BlockSpec((D, 16), lambda i: (0, 0)),
            pl.BlockSpec((16, D), lambda i: (0, 0)),
        ],
        out_specs=[ebs, ebs, pl.BlockSpec((EB, 16), lambda i: (i, 0))],
        out_shape=[
            jax.ShapeDtypeStruct((E, D), jnp.float32),
            jax.ShapeDtypeStruct((E, D), jnp.float32),
            jax.ShapeDtypeStruct((E, 16), jnp.float32),
        ],
    )(e, kg, qg, vg, We, WOe, _row(bOe), _row(g1e), _row(b1e), s16, t16)

    wv = jax.ops.segment_sum(vsc, dst, num_segments=N)
    z = jax.ops.segment_sum(scz, dst, num_segments=N)

    h2 = pl.pallas_call(
        _h2_body,
        grid=(N // NB,),
        in_specs=[
            pl.BlockSpec((NB, D), lambda i: (i, 0)),
            pl.BlockSpec((NB, 16), lambda i: (i, 0)),
            pl.BlockSpec((16, D), lambda i: (0, 0)),
            pl.BlockSpec((D, D), lambda i: (0, 0)),
            pl.BlockSpec((1, D), lambda i: (0, 0)),
            pl.BlockSpec((1, D), lambda i: (0, 0)),
            pl.BlockSpec((1, D), lambda i: (0, 0)),
        ],
        out_specs=pl.BlockSpec((NB, D), lambda i: (i, 0)),
        out_shape=jax.ShapeDtypeStruct((N, D), jnp.float32),
    )(wv, z, t16, WOh, _row(bOh), _row(g1h), _row(b1h))

    hs, hd = _sc_gather((h2, h2), (0, 1), src2, dst2)

    out = pl.pallas_call(
        _final_body,
        grid=(nb_e,),
        in_specs=[
            ebs, ebs, ebs,
            pl.BlockSpec((D, 192), lambda i: (0, 0)),
            pl.BlockSpec((D, 192), lambda i: (0, 0)),
            pl.BlockSpec((D, 192), lambda i: (0, 0)),
            pl.BlockSpec((1, 192), lambda i: (0, 0)),
            pl.BlockSpec((192, 4), lambda i: (0, 0)),
            pl.BlockSpec((1, 4), lambda i: (0, 0)),
        ],
        out_specs=pl.BlockSpec((EB, 4), lambda i: (i, 0)),
        out_shape=jax.ShapeDtypeStruct((E, 4), jnp.float32),
    )(hs, hd, e2, Wf0[0:D], Wf0[D:2 * D], Wf0[2 * D:3 * D],
      _row(bf0), Wf1, _row(bf1))
    return out


# speedup vs baseline: 14.7373x; 1.3611x over previous
import functools

import jax
import jax.numpy as jnp
import numpy as np
from jax.experimental import pallas as pl
from jax.experimental.pallas import tpu as pltpu
from jax.experimental.pallas import tpu_sc as plsc

N, E, D, H, DH = 10000, 320000, 128, 8, 16
NB = 1000
EB = 512
SCALE = 1.0 / np.sqrt(DH).astype(np.float32)

_S16 = np.zeros((D, 16), np.float32)
for _h in range(H):
    _S16[_h * DH:(_h + 1) * DH, _h] = 1.0
_T16 = _S16.T.copy()


def _qkv_body(h_ref, wq_ref, wk_ref, wv_ref, q_ref, k_ref, v_ref):
    hb = h_ref[...]
    q_ref[...] = jnp.dot(hb, wq_ref[...], preferred_element_type=jnp.float32)
    k_ref[...] = jnp.dot(hb, wk_ref[...], preferred_element_type=jnp.float32)
    v_ref[...] = jnp.dot(hb, wv_ref[...], preferred_element_type=jnp.float32)


def _pass1_body(e_ref, kg_ref, qg_ref, vg_ref, we_ref, woe_ref, boe_ref,
                g1e_ref, b1e_ref, s16_ref, t16_ref,
                e2_ref, vsc_ref, scb_ref):
    pe = jnp.dot(e_ref[...], we_ref[...], preferred_element_type=jnp.float32)
    score = kg_ref[...] * qg_ref[...] * pe * SCALE
    sc16 = jnp.exp(jnp.clip(
        jnp.dot(score, s16_ref[...], preferred_element_type=jnp.float32),
        -5.0, 5.0))
    scb = jnp.dot(sc16, t16_ref[...], preferred_element_type=jnp.float32)
    scb_ref[...] = scb
    vsc_ref[...] = vg_ref[...] * scb
    t = jnp.dot(score, woe_ref[...], preferred_element_type=jnp.float32) \
        + boe_ref[...]
    m = jnp.mean(t, axis=1, keepdims=True)
    v = jnp.mean((t - m) * (t - m), axis=1, keepdims=True)
    e2_ref[...] = (t - m) * jax.lax.rsqrt(v + 1e-5) * g1e_ref[...] \
        + b1e_ref[...]


def _h2_body(wv_ref, zb_ref, woh_ref, boh_ref,
             g1h_ref, b1h_ref, h2_ref):
    h_out = wv_ref[...] / (zb_ref[...] + 1e-6)
    t = jnp.dot(h_out, woh_ref[...], preferred_element_type=jnp.float32) \
        + boh_ref[...]
    m = jnp.mean(t, axis=1, keepdims=True)
    v = jnp.mean((t - m) * (t - m), axis=1, keepdims=True)
    h2_ref[...] = (t - m) * jax.lax.rsqrt(v + 1e-5) * g1h_ref[...] \
        + b1h_ref[...]


def _final_body(hs_ref, hd_ref, e2_ref, wtop_ref, wmid_ref, wbot_ref,
                bf0_ref, wf1_ref, bf1_ref, out_ref):
    y = (jnp.dot(hs_ref[...], wtop_ref[...],
                 preferred_element_type=jnp.float32)
         + jnp.dot(e2_ref[...], wmid_ref[...],
                   preferred_element_type=jnp.float32)
         + jnp.dot(hd_ref[...], wbot_ref[...],
                   preferred_element_type=jnp.float32)
         + bf0_ref[...])
    y = jnp.maximum(y, 0.0)
    out_ref[...] = jnp.dot(y, wf1_ref[...],
                           preferred_element_type=jnp.float32) + bf1_ref[...]


def _row(x):
    return x.reshape(1, -1)


_NC, _NS = 2, 16
_NW = _NC * _NS
_RPW = E // _NW
_CH = 80
_NCH = _RPW // _CH


def _sc_mesh():
    return plsc.VectorSubcoreMesh(core_axis_name="c", subcore_axis_name="s",
                                  num_cores=_NC, num_subcores=_NS)


def _make_gather_body(sels):
    n_t = len(sels)

    def body(*refs):
        tables = refs[:n_t]
        src2, dst2 = refs[n_t], refs[n_t + 1]
        outs = refs[n_t + 2:2 * n_t + 2]
        idx_s, idx_d = refs[2 * n_t + 2], refs[2 * n_t + 3]
        bufs = refs[2 * n_t + 4:3 * n_t + 4]
        sems = refs[3 * n_t + 4:]
        w = jax.lax.axis_index("s") * _NC + jax.lax.axis_index("c")
        base = w * _NCH
        pltpu.sync_copy(src2.at[w], idx_s)
        pltpu.sync_copy(dst2.at[w], idx_d)
        idxs = (idx_s, idx_d)

        def step(j, carry):
            cps = [pltpu.async_copy(tables[t].at[idxs[sels[t]].at[j]],
                                    bufs[t], sems[t]) for t in range(n_t)]
            for cp in cps:
                cp.wait()
            row0 = (base + j) * _CH
            for t in range(n_t):
                pltpu.sync_copy(bufs[t], outs[t].at[pl.ds(row0, _CH)])
            return carry

        jax.lax.fori_loop(0, _NCH, step, 0)

    return body


_NCH_S = E // (_CH * _NS)


def _scatter_body(vsc_hbm, scb_hbm, dst2, wv_out, zb_out,
                  shared, idxrow, vbuf):
    c = jax.lax.axis_index("c")
    s = jax.lax.axis_index("s")
    zv = jnp.zeros((16,), jnp.float32)

    def fill(i, carry):
        for j in range(D // 16):
            vbuf[i, pl.ds(j * 16, 16)] = zv
        return carry

    jax.lax.fori_loop(0, _CH, fill, 0)

    c0 = s * (N // _CH) // _NS
    c1 = (s + 1) * (N // _CH) // _NS

    def iota_rows(i):
        for k in range(_CH // 16):
            idxrow[pl.ds(k * 16, 16)] = (jax.lax.iota(jnp.int32, 16)
                                         + i * _CH + k * 16)

    def zstep(i, carry):
        iota_rows(i)
        pltpu.sync_copy(vbuf, shared.at[idxrow])
        return carry

    jax.lax.fori_loop(c0, c1, zstep, 0)
    plsc.subcore_barrier()

    def make_step(payload):
        def step(j, carry):
            row0 = (s * _NCH_S + j) * _CH
            pltpu.sync_copy(payload.at[pl.ds(row0, _CH)], vbuf)
            pltpu.sync_copy(dst2.at[s].at[j], idxrow)
            pltpu.sync_copy(vbuf, shared.at[idxrow], add=True)
            return carry
        return step

    @pl.when(c == 0)
    def _():
        jax.lax.fori_loop(0, _NCH_S, make_step(vsc_hbm), 0)

    @pl.when(c == 1)
    def _():
        jax.lax.fori_loop(0, _NCH_S, make_step(scb_hbm), 0)

    plsc.subcore_barrier()

    def make_ostep(out):
        def ostep(i, carry):
            iota_rows(i)
            pltpu.sync_copy(shared.at[idxrow], vbuf)
            pltpu.sync_copy(vbuf, out.at[pl.ds(i * _CH, _CH)])
            return carry
        return ostep

    @pl.when(c == 0)
    def _():
        jax.lax.fori_loop(c0, c1, make_ostep(wv_out), 0)

    @pl.when(c == 1)
    def _():
        jax.lax.fori_loop(c0, c1, make_ostep(zb_out), 0)


def _sc_scatter(vsc, scb, dst2):
    k = pl.kernel(
        _scatter_body,
        out_type=[jax.ShapeDtypeStruct((N, D), jnp.float32),
                  jax.ShapeDtypeStruct((N, D), jnp.float32)],
        mesh=_sc_mesh(),
        scratch_types=[
            pltpu.VMEM_SHARED((N, D), jnp.float32),
            pltpu.VMEM((_CH,), jnp.int32),
            pltpu.VMEM((_CH, D), jnp.float32),
        ],
    )
    return k(vsc, scb, dst2)


def _sc_gather(tables, sels, src2, dst2):
    n_t = len(tables)
    d_cols = [int(t.shape[1]) for t in tables]
    scratch = (
        [pltpu.VMEM((_NCH, _CH), jnp.int32)] * 2
        + [pltpu.VMEM((_CH, d_cols[t]), jnp.float32) for t in range(n_t)]
        + [pltpu.SemaphoreType.DMA] * n_t)
    k = pl.kernel(
        _make_gather_body(sels),
        out_type=[jax.ShapeDtypeStruct((E, d_cols[t]), jnp.float32)
                  for t in range(n_t)],
        mesh=_sc_mesh(),
        scratch_types=scratch,
    )
    return k(*tables, src2, dst2)


def kernel(h, e, edge_index, WQ, WK, WV, We, WOh, bOh, WOe, bOe,
           g1h, b1h, g1e, b1e, Wf0, bf0, Wf1, bf1):
    src = edge_index[0]
    dst = edge_index[1]
    src2 = src.reshape(_NW, _NCH, _CH)
    dst2 = dst.reshape(_NW, _NCH, _CH)
    s16 = jnp.asarray(_S16)
    t16 = jnp.asarray(_T16)

    wspec = pl.BlockSpec((D, D), lambda i: (0, 0))
    q_t, k_t, v_t = pl.pallas_call(
        _qkv_body,
        grid=(N // NB,),
        in_specs=[pl.BlockSpec((NB, D), lambda i: (i, 0)), wspec, wspec,
                  wspec],
        out_specs=[pl.BlockSpec((NB, D), lambda i: (i, 0))] * 3,
        out_shape=[jax.ShapeDtypeStruct((N, D), jnp.float32)] * 3,
    )(h, WQ, WK, WV)

    kg, qg, vg = _sc_gather((k_t, q_t, v_t), (0, 1, 0), src2, dst2)

    nb_e = E // EB
    ebs = pl.BlockSpec((EB, D), lambda i: (i, 0))
    e2, vsc, scb = pl.pallas_call(
        _pass1_body,
        grid=(nb_e,),
        in_specs=[
            pl.BlockSpec((EB, D + 2), lambda i: (i, 0)),
            ebs, ebs, ebs,
            pl.BlockSpec((D + 2, D), lambda i: (0, 0)),
            pl.BlockSpec((D, D), lambda i: (0, 0)),
            pl.BlockSpec((1, D), lambda i: (0, 0)),
            pl.BlockSpec((1, D), lambda i: (0, 0)),
            pl.BlockSpec((1, D), lambda i: (0, 0)),
            pl.BlockSpec((D, 16), lambda i: (0, 0)),
            pl.BlockSpec((16, D), lambda i: (0, 0)),
        ],
        out_specs=[ebs, ebs, ebs],
        out_shape=[
            jax.ShapeDtypeStruct((E, D), jnp.float32),
            jax.ShapeDtypeStruct((E, D), jnp.float32),
            jax.ShapeDtypeStruct((E, D), jnp.float32),
        ],
    )(e, kg, qg, vg, We, WOe, _row(bOe), _row(g1e), _row(b1e), s16, t16)

    dst2s = dst.reshape(_NS, _NCH_S, _CH)
    wv, zb = _sc_scatter(vsc, scb, dst2s)

    h2 = pl.pallas_call(
        _h2_body,
        grid=(N // NB,),
        in_specs=[
            pl.BlockSpec((NB, D), lambda i: (i, 0)),
            pl.BlockSpec((NB, D), lambda i: (i, 0)),
            pl.BlockSpec((D, D), lambda i: (0, 0)),
            pl.BlockSpec((1, D), lambda i: (0, 0)),
            pl.BlockSpec((1, D), lambda i: (0, 0)),
            pl.BlockSpec((1, D), lambda i: (0, 0)),
        ],
        out_specs=pl.BlockSpec((NB, D), lambda i: (i, 0)),
        out_shape=jax.ShapeDtypeStruct((N, D), jnp.float32),
    )(wv, zb, WOh, _row(bOh), _row(g1h), _row(b1h))

    hs, hd = _sc_gather((h2, h2), (0, 1), src2, dst2)

    out = pl.pallas_call(
        _final_body,
        grid=(nb_e,),
        in_specs=[
            ebs, ebs, ebs,
            pl.BlockSpec((D, 192), lambda i: (0, 0)),
            pl.BlockSpec((D, 192), lambda i: (0, 0)),
            pl.BlockSpec((D, 192), lambda i: (0, 0)),
            pl.BlockSpec((1, 192), lambda i: (0, 0)),
            pl.BlockSpec((192, 4), lambda i: (0, 0)),
            pl.BlockSpec((1, 4), lambda i: (0, 0)),
        ],
        out_specs=pl.BlockSpec((EB, 4), lambda i: (i, 0)),
        out_shape=jax.ShapeDtypeStruct((E, 4), jnp.float32),
    )(hs, hd, e2, Wf0[0:D], Wf0[D:2 * D], Wf0[2 * D:3 * D],
      _row(bf0), Wf1, _row(bf1))
    return out
